# Initial kernel scaffold; baseline (speedup 1.0000x reference)
import jax
import jax.numpy as jnp
from jax.experimental import pallas as pl


def kernel(spike_trains, W, b):
    raise NotImplementedError("write your pallas kernel here")


# speedup vs baseline: 1.0023x; 1.0023x over previous
import jax
import jax.numpy as jnp
from jax.experimental import pallas as pl
from jax.experimental.pallas import tpu as pltpu

_TAU_DECAY = 10.0


def _body(d_ref, s_ref, w_ref, b_ref, o_ref):
    s = s_ref[0]
    d = d_ref[...]
    fr = jax.lax.dot_general(
        d, s, (((1,), (0,)), ((), ())), preferred_element_type=jnp.float32
    )
    out = jax.lax.dot_general(
        fr, w_ref[...], (((1,), (1,)), ((), ())),
        preferred_element_type=jnp.float32,
    )
    o_ref[0] = out + b_ref[...]


def kernel(spike_trains, W, b):
    B, T, N = spike_trains.shape
    O = W.shape[0]
    decay = jnp.exp(-jnp.arange(T, dtype=spike_trains.dtype) / _TAU_DECAY)
    decay = (decay / decay.sum()).reshape(1, T)
    b2 = b.reshape(1, O)
    return pl.pallas_call(
        _body,
        grid=(B,),
        in_specs=[
            pl.BlockSpec((1, T), lambda i: (0, 0)),
            pl.BlockSpec((1, T, N), lambda i: (i, 0, 0)),
            pl.BlockSpec((O, N), lambda i: (0, 0)),
            pl.BlockSpec((1, O), lambda i: (0, 0)),
        ],
        out_specs=pl.BlockSpec((1, 1, O), lambda i: (i, 0, 0)),
        out_shape=jax.ShapeDtypeStruct((B, 1, O), spike_trains.dtype),
        compiler_params=pltpu.CompilerParams(
            dimension_semantics=("parallel",),
        ),
        name="spike_rate_readout",
    )(decay, spike_trains, W, b2).reshape(B, O)
